# baseline (speedup 1.0000x reference)
import jax
import jax.numpy as jnp
from jax import lax
from jax.experimental import pallas as pl
from jax.experimental.pallas import tpu as pltpu
from jax.experimental.pallas import tpu_sc as plsc

NUM_SEG = 64
D = 128
N_ROWS = 100000
NC, NS = 2, 16
NW = NC * NS
CHUNK_R = 128
SC_K = 14
SC_ROWS = NW * SC_K * CHUNK_R
TC_BLK = 1024
TC_GRID = (N_ROWS - SC_ROWS) // TC_BLK
TAIL0 = SC_ROWS + TC_GRID * TC_BLK
IDS_MAIN = SC_K * CHUNK_R
NEG_INF = float("-inf")


def _sc_body(x_hbm, ids_hbm, out_hbm, idsv, idst, bufx0, bufx1, acc,
             sem0, sem1):
    cc = lax.axis_index("c")
    ss = lax.axis_index("s")
    wid = ss * NC + cc

    base_row = wid * IDS_MAIN
    tail_start = jnp.minimum(TAIL0 + wid * CHUNK_R, N_ROWS - CHUNK_R)

    pltpu.sync_copy(ids_hbm.at[pl.ds(base_row, IDS_MAIN)],
                    idsv.at[pl.ds(0, IDS_MAIN)])
    pltpu.sync_copy(ids_hbm.at[pl.ds(tail_start, CHUNK_R)],
                    idst.at[pl.ds(0, CHUNK_R)])

    def ini(i, carry):
        for j in range(D // 16):
            acc[i, pl.ds(j * 16, 16)] = jnp.full((16,), NEG_INF, jnp.float32)
        return carry
    lax.fori_loop(0, NUM_SEG, ini, 0)

    def start_of(cs):
        return jnp.where(cs < SC_K, base_row + cs * CHUNK_R, tail_start)

    def issue(cs, bufx, sem):
        pltpu.async_copy(x_hbm.at[pl.ds(start_of(cs), CHUNK_R), :],
                         bufx, sem)

    def wait(bufx, sem):
        pltpu.make_async_copy(x_hbm.at[pl.ds(0, CHUNK_R), :],
                              bufx, sem).wait()

    lane = lax.iota(jnp.int32, 16)

    def process(bufx, idref, off):
        idv0 = idref[pl.ds(off, 16)]
        idvl = idref[pl.ds(off + CHUNK_R - 16, 16)]

        def fast_chunk(_):
            neg = jnp.full((16,), NEG_INF, jnp.float32)

            def gb(g, m):
                out = list(m)
                for j in range(D // 16):
                    sl = pl.ds(j * 16, 16)
                    vals = [bufx[g * 16 + k, sl] for k in range(16)]
                    while len(vals) > 1:
                        vals = [jnp.maximum(vals[2 * t], vals[2 * t + 1])
                                for t in range(len(vals) // 2)]
                    out[j] = jnp.maximum(out[j], vals[0])
                return tuple(out)

            red = lax.fori_loop(0, CHUNK_R // 16, gb, (neg,) * (D // 16))
            for j in range(D // 16):
                colv = lane + (j * 16)
                cur = plsc.load_gather(acc, [idv0, colv])
                plsc.store_scatter(acc, [idv0, colv],
                                   jnp.maximum(cur, red[j]))
            return 0

        def slow_chunk(_):
            def group_body(gi, carry):
                o = off + gi * 16
                idv = idref[pl.ds(o, 16)]
                i0 = idv[0]
                i15 = idv[15]

                def fast(_):
                    for j in range(D // 16):
                        sl = pl.ds(j * 16, 16)
                        vals = [bufx[gi * 16 + k, sl] for k in range(16)]
                        while len(vals) > 1:
                            vals = [jnp.maximum(vals[2 * t], vals[2 * t + 1])
                                    for t in range(len(vals) // 2)]
                        acc[i0, sl] = jnp.maximum(acc[i0, sl], vals[0])
                    return 0

                def slow(_):
                    def rb(k, c2):
                        seg = idref[pl.ds(o + k, 16)][0]
                        for j in range(D // 16):
                            sl = pl.ds(j * 16, 16)
                            acc[seg, sl] = jnp.maximum(acc[seg, sl],
                                                       bufx[gi * 16 + k, sl])
                        return c2
                    return lax.fori_loop(0, 16, rb, 0)

                lax.cond(i0 == i15, fast, slow, 0)
                return carry

            return lax.fori_loop(0, CHUNK_R // 16, group_body, 0)

        lax.cond(idv0[0] == idvl[15], fast_chunk, slow_chunk, 0)

    issue(0, bufx0, sem0)

    def pair(p, carry):
        c0 = 2 * p
        issue(c0 + 1, bufx1, sem1)
        wait(bufx0, sem0)
        process(bufx0, idsv, c0 * CHUNK_R)
        issue(c0 + 2, bufx0, sem0)
        wait(bufx1, sem1)
        process(bufx1, idsv, (c0 + 1) * CHUNK_R)
        return carry

    lax.fori_loop(0, SC_K // 2, pair, 0)
    wait(bufx0, sem0)
    process(bufx0, idst, 0)
    pltpu.sync_copy(acc, out_hbm.at[wid])


def _sc_partials(x, ids):
    mesh = plsc.VectorSubcoreMesh(core_axis_name="c", subcore_axis_name="s")
    return pl.kernel(
        _sc_body,
        out_type=jax.ShapeDtypeStruct((NW, NUM_SEG, D), jnp.float32),
        mesh=mesh,
        compiler_params=pltpu.CompilerParams(use_tc_tiling_on_sc=False,
                                             needs_layout_passes=False),
        scratch_types=[
            pltpu.VMEM((IDS_MAIN + 16,), jnp.int32),
            pltpu.VMEM((CHUNK_R + 16,), jnp.int32),
            pltpu.VMEM((CHUNK_R, D), jnp.float32),
            pltpu.VMEM((CHUNK_R, D), jnp.float32),
            pltpu.VMEM((NUM_SEG, D), jnp.float32),
            pltpu.SemaphoreType.DMA,
            pltpu.SemaphoreType.DMA,
        ],
    )(x, ids)


NB = TC_BLK // 8


def _tc_pool_body(x_ref, ids_ref, o_ref, bm_ref):
    i = pl.program_id(0)

    @pl.when(i == 0)
    def _():
        o_ref[...] = jnp.full((NUM_SEG, D), NEG_INF, jnp.float32)

    idb = ids_ref[0]
    lo = jnp.min(idb)
    hi = jnp.max(idb)

    for s in range(TC_BLK // 128):
        sub = x_ref[pl.ds(s * 128, 128), :].reshape(16, 8, D)
        bm_ref[pl.ds(s * 16, 16), :] = jnp.max(sub, axis=1)

    biota = lax.broadcasted_iota(jnp.int32, (NB, 1), 0)
    riota8 = lax.broadcasted_iota(jnp.int32, (8, 1), 0)

    def gbody(g, carry):
        a = jnp.sum((idb < g).astype(jnp.int32))
        b_ = jnp.sum((idb <= g).astype(jnp.int32))
        t0 = (a + 7) // 8
        t1 = b_ // 8
        mc = jnp.max(jnp.where((biota >= t0) & (biota < t1),
                               bm_ref[...], NEG_INF), axis=0, keepdims=True)
        ha = jnp.minimum((a // 8) * 8, TC_BLK - 8)
        hv = x_ref[pl.ds(ha, 8), :]
        hm = jnp.max(jnp.where(((riota8 + ha) >= a) & ((riota8 + ha) < b_),
                               hv, NEG_INF), axis=0, keepdims=True)
        ta = jnp.minimum(t1 * 8, TC_BLK - 8)
        tv = x_ref[pl.ds(ta, 8), :]
        tm = jnp.max(jnp.where(((riota8 + ta) >= a) & ((riota8 + ta) < b_),
                               tv, NEG_INF), axis=0, keepdims=True)
        m = jnp.maximum(jnp.maximum(mc, hm), tm)
        o_ref[pl.ds(g, 1), :] = jnp.maximum(o_ref[pl.ds(g, 1), :], m)
        return carry

    lax.fori_loop(lo, hi + 1, gbody, 0)


def _tc_partials(x, ids_blocks):
    return pl.pallas_call(
        _tc_pool_body,
        grid=(TC_GRID,),
        in_specs=[
            pl.BlockSpec((TC_BLK, D), lambda i: (SC_ROWS // TC_BLK + i, 0)),
            pl.BlockSpec((1, 1, TC_BLK), lambda i: (i, 0, 0)),
        ],
        out_specs=pl.BlockSpec((NUM_SEG, D), lambda i: (0, 0)),
        out_shape=jax.ShapeDtypeStruct((NUM_SEG, D), jnp.float32),
        scratch_shapes=[pltpu.VMEM((NB, D), jnp.float32)],
    )(x, ids_blocks)


def _combine_body(psc_ref, ptc_ref, w_ref, b_ref, o_ref):
    h = jnp.maximum(jnp.max(psc_ref[...], axis=0), ptc_ref[...])
    y = lax.dot_general(h, w_ref[...], (((1,), (1,)), ((), ())),
                        preferred_element_type=jnp.float32)
    o_ref[...] = jnp.maximum(y + b_ref[...], 0.0)


def _combine_linear(p_sc, p_tc, W, b2d):
    return pl.pallas_call(
        _combine_body,
        out_shape=jax.ShapeDtypeStruct((NUM_SEG, D), jnp.float32),
    )(p_sc, p_tc, W, b2d)


def kernel(x, edge_index, batch, W, b):
    del edge_index
    ids = batch.astype(jnp.int32)
    ids_blocks = ids[SC_ROWS:TAIL0].reshape(TC_GRID, 1, TC_BLK)
    p_tc = _tc_partials(x, ids_blocks)
    p_sc = _sc_partials(x, ids)
    return _combine_linear(p_sc, p_tc, W, b.reshape(1, D))


# speedup vs baseline: 1.1338x; 1.1338x over previous
import jax
import jax.numpy as jnp
from jax import lax
from jax.experimental import pallas as pl
from jax.experimental.pallas import tpu as pltpu
from jax.experimental.pallas import tpu_sc as plsc

NUM_SEG = 64
D = 128
N_ROWS = 100000
NC, NS = 2, 16
NW = NC * NS
CHUNK_R = 128
SC_K = 16
SC_ROWS = NW * SC_K * CHUNK_R
TC_BLK = 1024
TC_GRID = (N_ROWS - SC_ROWS) // TC_BLK
TAIL0 = SC_ROWS + TC_GRID * TC_BLK
IDS_MAIN = SC_K * CHUNK_R
NEG_INF = float("-inf")


def _sc_body(x_hbm, ids_hbm, out_hbm, idsv, idst, bufx0, bufx1, acc,
             sem0, sem1):
    cc = lax.axis_index("c")
    ss = lax.axis_index("s")
    wid = ss * NC + cc

    base_row = wid * IDS_MAIN
    tail_start = jnp.minimum(TAIL0 + wid * CHUNK_R, N_ROWS - CHUNK_R)

    pltpu.sync_copy(ids_hbm.at[pl.ds(base_row, IDS_MAIN)],
                    idsv.at[pl.ds(0, IDS_MAIN)])
    pltpu.sync_copy(ids_hbm.at[pl.ds(tail_start, CHUNK_R)],
                    idst.at[pl.ds(0, CHUNK_R)])

    def ini(i, carry):
        for j in range(D // 16):
            acc[i, pl.ds(j * 16, 16)] = jnp.full((16,), NEG_INF, jnp.float32)
        return carry
    lax.fori_loop(0, NUM_SEG, ini, 0)

    def start_of(cs):
        return jnp.where(cs < SC_K, base_row + cs * CHUNK_R, tail_start)

    def issue(cs, bufx, sem):
        pltpu.async_copy(x_hbm.at[pl.ds(start_of(cs), CHUNK_R), :],
                         bufx, sem)

    def wait(bufx, sem):
        pltpu.make_async_copy(x_hbm.at[pl.ds(0, CHUNK_R), :],
                              bufx, sem).wait()

    lane = lax.iota(jnp.int32, 16)

    def process(bufx, idref, off):
        idv0 = idref[pl.ds(off, 16)]
        idvl = idref[pl.ds(off + CHUNK_R - 16, 16)]

        def fast_chunk(_):
            neg = jnp.full((16,), NEG_INF, jnp.float32)

            def gb(g, m):
                out = list(m)
                for j in range(D // 16):
                    sl = pl.ds(j * 16, 16)
                    vals = [bufx[g * 16 + k, sl] for k in range(16)]
                    while len(vals) > 1:
                        vals = [jnp.maximum(vals[2 * t], vals[2 * t + 1])
                                for t in range(len(vals) // 2)]
                    out[j] = jnp.maximum(out[j], vals[0])
                return tuple(out)

            red = lax.fori_loop(0, CHUNK_R // 16, gb, (neg,) * (D // 16))
            for j in range(D // 16):
                colv = lane + (j * 16)
                cur = plsc.load_gather(acc, [idv0, colv])
                plsc.store_scatter(acc, [idv0, colv],
                                   jnp.maximum(cur, red[j]))
            return 0

        def slow_chunk(_):
            def group_body(gi, carry):
                o = off + gi * 16
                idv = idref[pl.ds(o, 16)]
                i0 = idv[0]
                i15 = idv[15]

                def fast(_):
                    for j in range(D // 16):
                        sl = pl.ds(j * 16, 16)
                        vals = [bufx[gi * 16 + k, sl] for k in range(16)]
                        while len(vals) > 1:
                            vals = [jnp.maximum(vals[2 * t], vals[2 * t + 1])
                                    for t in range(len(vals) // 2)]
                        acc[i0, sl] = jnp.maximum(acc[i0, sl], vals[0])
                    return 0

                def slow(_):
                    def rb(k, c2):
                        seg = idref[pl.ds(o + k, 16)][0]
                        for j in range(D // 16):
                            sl = pl.ds(j * 16, 16)
                            acc[seg, sl] = jnp.maximum(acc[seg, sl],
                                                       bufx[gi * 16 + k, sl])
                        return c2
                    return lax.fori_loop(0, 16, rb, 0)

                lax.cond(i0 == i15, fast, slow, 0)
                return carry

            return lax.fori_loop(0, CHUNK_R // 16, group_body, 0)

        lax.cond(idv0[0] == idvl[15], fast_chunk, slow_chunk, 0)

    issue(0, bufx0, sem0)

    def pair(p, carry):
        c0 = 2 * p
        issue(c0 + 1, bufx1, sem1)
        wait(bufx0, sem0)
        process(bufx0, idsv, c0 * CHUNK_R)
        issue(c0 + 2, bufx0, sem0)
        wait(bufx1, sem1)
        process(bufx1, idsv, (c0 + 1) * CHUNK_R)
        return carry

    lax.fori_loop(0, SC_K // 2, pair, 0)
    wait(bufx0, sem0)
    process(bufx0, idst, 0)
    pltpu.sync_copy(acc, out_hbm.at[wid])


def _sc_partials(x, ids):
    mesh = plsc.VectorSubcoreMesh(core_axis_name="c", subcore_axis_name="s")
    return pl.kernel(
        _sc_body,
        out_type=jax.ShapeDtypeStruct((NW, NUM_SEG, D), jnp.float32),
        mesh=mesh,
        compiler_params=pltpu.CompilerParams(use_tc_tiling_on_sc=False,
                                             needs_layout_passes=False),
        scratch_types=[
            pltpu.VMEM((IDS_MAIN + 16,), jnp.int32),
            pltpu.VMEM((CHUNK_R + 16,), jnp.int32),
            pltpu.VMEM((CHUNK_R, D), jnp.float32),
            pltpu.VMEM((CHUNK_R, D), jnp.float32),
            pltpu.VMEM((NUM_SEG, D), jnp.float32),
            pltpu.SemaphoreType.DMA,
            pltpu.SemaphoreType.DMA,
        ],
    )(x, ids)


NB = TC_BLK // 8


def _tc_pool_body(x_ref, ids_ref, o_ref, bm_ref):
    i = pl.program_id(0)

    @pl.when(i == 0)
    def _():
        o_ref[...] = jnp.full((NUM_SEG, D), NEG_INF, jnp.float32)

    idb = ids_ref[...]
    lo = jnp.min(idb)
    hi = jnp.max(idb)

    for s in range(TC_BLK // 128):
        sub = x_ref[pl.ds(s * 128, 128), :].reshape(16, 8, D)
        bm_ref[pl.ds(s * 16, 16), :] = jnp.max(sub, axis=1)

    biota = lax.broadcasted_iota(jnp.int32, (NB, 1), 0)
    riota8 = lax.broadcasted_iota(jnp.int32, (8, 1), 0)

    def gbody(g, carry):
        a = jnp.sum((idb < g).astype(jnp.int32))
        b_ = jnp.sum((idb <= g).astype(jnp.int32))
        t0 = (a + 7) // 8
        t1 = b_ // 8
        mc = jnp.max(jnp.where((biota >= t0) & (biota < t1),
                               bm_ref[...], NEG_INF), axis=0, keepdims=True)
        ha = jnp.minimum((a // 8) * 8, TC_BLK - 8)
        hv = x_ref[pl.ds(ha, 8), :]
        hm = jnp.max(jnp.where(((riota8 + ha) >= a) & ((riota8 + ha) < b_),
                               hv, NEG_INF), axis=0, keepdims=True)
        ta = jnp.minimum(t1 * 8, TC_BLK - 8)
        tv = x_ref[pl.ds(ta, 8), :]
        tm = jnp.max(jnp.where(((riota8 + ta) >= a) & ((riota8 + ta) < b_),
                               tv, NEG_INF), axis=0, keepdims=True)
        m = jnp.maximum(jnp.maximum(mc, hm), tm)
        o_ref[pl.ds(g, 1), :] = jnp.maximum(o_ref[pl.ds(g, 1), :], m)
        return carry

    lax.fori_loop(lo, hi + 1, gbody, 0)


def _tc_partials(x, ids):
    return pl.pallas_call(
        _tc_pool_body,
        grid=(TC_GRID,),
        in_specs=[
            pl.BlockSpec((TC_BLK, D), lambda i: (SC_ROWS // TC_BLK + i, 0)),
            pl.BlockSpec((TC_BLK,), lambda i: (SC_ROWS // TC_BLK + i,)),
        ],
        out_specs=pl.BlockSpec((NUM_SEG, D), lambda i: (0, 0)),
        out_shape=jax.ShapeDtypeStruct((NUM_SEG, D), jnp.float32),
        scratch_shapes=[pltpu.VMEM((NB, D), jnp.float32)],
    )(x, ids)


def _combine_body(psc_ref, ptc_ref, w_ref, b_ref, o_ref):
    h = jnp.maximum(jnp.max(psc_ref[...], axis=0), ptc_ref[...])
    y = lax.dot_general(h, w_ref[...], (((1,), (1,)), ((), ())),
                        preferred_element_type=jnp.float32)
    o_ref[...] = jnp.maximum(y + b_ref[...], 0.0)


def _combine_linear(p_sc, p_tc, W, b2d):
    return pl.pallas_call(
        _combine_body,
        out_shape=jax.ShapeDtypeStruct((NUM_SEG, D), jnp.float32),
    )(p_sc, p_tc, W, b2d)


def kernel(x, edge_index, batch, W, b):
    del edge_index
    ids = batch.astype(jnp.int32)
    p_tc = _tc_partials(x, ids)
    p_sc = _sc_partials(x, ids)
    return _combine_linear(p_sc, p_tc, W, b.reshape(1, D))
